# baseline (speedup 1.0000x reference)
import functools

import jax
import jax.numpy as jnp
from jax import lax
from jax.experimental import pallas as pl
from jax.experimental.pallas import tpu as pltpu
from jax.experimental.pallas import tpu_sc as plsc

N = 10000
E = 320000
D = 128
DO = 64
DH = 64

NC = 2
NS = 16
NW = NC * NS
L = 16

EPC = E // NS
CH = 128
NCHS = -(-EPC // CH)
EPC_PAD = NCHS * CH
ZROWS = 632
ACC_N = NS * ZROWS
OROWS = 624
TAIL = N - NS * OROWS

EPT = E // NW
DCH = 80
DNCH = EPT // DCH

_MESH = plsc.VectorSubcoreMesh(
    core_axis_name="c", subcore_axis_name="s", num_cores=NC, num_subcores=NS)


def _wid():
    return lax.axis_index("c") * NS + lax.axis_index("s")


@functools.partial(
    pl.kernel,
    out_type=jax.ShapeDtypeStruct((NW * N,), jnp.float32),
    mesh=_MESH,
    compiler_params=pltpu.CompilerParams(needs_layout_passes=False, use_tc_tiling_on_sc=False),
    scratch_types=[
        pltpu.VMEM((EPT,), jnp.int32),
        pltpu.VMEM((N,), jnp.float32),
    ],
)
def _deg_kernel(dst_hbm, zeros_hbm, out_hbm, dst_v, hist_v):
    w = _wid()
    pltpu.sync_copy(dst_hbm.at[pl.ds(w * EPT, EPT)], dst_v)
    pltpu.sync_copy(zeros_hbm, hist_v)
    ones = jnp.full((L,), 1.0, dtype=jnp.float32)

    def body(i, carry):
        idx = dst_v[pl.ds(i * L, L)]
        plsc.addupdate_scatter(hist_v, [idx], ones)
        return carry

    lax.fori_loop(0, EPT // L, body, 0)
    pltpu.sync_copy(hist_v, out_hbm.at[pl.ds(w * N, N)])


@functools.partial(
    pl.kernel,
    out_type=jax.ShapeDtypeStruct((NC, N, DH), jnp.float32),
    mesh=_MESH,
    compiler_params=pltpu.CompilerParams(use_tc_tiling_on_sc=False),
    scratch_types=[
        pltpu.VMEM((NCHS, CH), jnp.int32),
        pltpu.VMEM((NCHS, CH), jnp.int32),
        pltpu.VMEM((CH, DH), jnp.float32),
        pltpu.VMEM((CH, DH), jnp.float32),
        pltpu.VMEM_SHARED((ACC_N, DH), jnp.float32),
        pltpu.SemaphoreType.DMA,
        pltpu.SemaphoreType.DMA,
        pltpu.SemaphoreType.DMA,
        pltpu.SemaphoreType.DMA,
    ],
)
def _scatter_kernel(vs_hbm, srcp_hbm, dstp_hbm, zslab_hbm, out_hbm,
                    src_v, dst_v, rows0, rows1, acc_sh,
                    gsem0, gsem1, ssem0, ssem1):
    c = lax.axis_index("c")
    s = lax.axis_index("s")
    pltpu.sync_copy(zslab_hbm, acc_sh.at[pl.ds(s * ZROWS, ZROWS)])
    pltpu.sync_copy(srcp_hbm.at[s], src_v)
    pltpu.sync_copy(dstp_hbm.at[s], dst_v)
    plsc.subcore_barrier()

    vhalf = vs_hbm.at[c]
    bufs = (rows0, rows1)
    gsems = (gsem0, gsem1)
    ssems = (ssem0, ssem1)
    gdesc = [None, None]
    sdesc = [None, None]
    gdesc[0] = pltpu.async_copy(vhalf.at[src_v.at[0]], rows0, gsem0)
    for j in range(NCHS):
        p = j % 2
        q = (j + 1) % 2
        if j + 1 < NCHS:
            if j >= 1:
                sdesc[q].wait()
            gdesc[q] = pltpu.async_copy(
                vhalf.at[src_v.at[j + 1]], bufs[q], gsems[q])
        gdesc[p].wait()
        sdesc[p] = pltpu.async_copy(
            bufs[p], acc_sh.at[dst_v.at[j]], ssems[p], add=True)
    sdesc[0].wait()
    sdesc[1].wait()

    plsc.subcore_barrier()
    pltpu.sync_copy(acc_sh.at[pl.ds(s * OROWS, OROWS)],
                    out_hbm.at[c, pl.ds(s * OROWS, OROWS)])

    @pl.when(s == NS - 1)
    def _tail():
        pltpu.sync_copy(acc_sh.at[pl.ds(NS * OROWS, TAIL)],
                        out_hbm.at[c, pl.ds(NS * OROWS, TAIL)])


@functools.partial(
    pl.kernel,
    out_type=jax.ShapeDtypeStruct((E,), jnp.float32),
    mesh=_MESH,
    compiler_params=pltpu.CompilerParams(needs_layout_passes=False, use_tc_tiling_on_sc=False),
    scratch_types=[
        pltpu.VMEM((DNCH, DCH), jnp.int32),
        pltpu.VMEM((DNCH, DCH), jnp.int32),
        pltpu.VMEM((DCH, DO), jnp.float32),
        pltpu.VMEM((DCH, DO), jnp.float32),
        pltpu.VMEM((DCH, DO), jnp.float32),
        pltpu.VMEM((DCH, DO), jnp.float32),
        pltpu.VMEM((EPT,), jnp.float32),
        pltpu.VMEM_SHARED((N, DO), jnp.float32),
        pltpu.SemaphoreType.DMA,
        pltpu.SemaphoreType.DMA,
        pltpu.SemaphoreType.DMA,
        pltpu.SemaphoreType.DMA,
    ],
)
def _decode_kernel(z_hbm, e0_hbm, e1_hbm, out_hbm,
                   e0_v, e1_v, zi_a, zj_a, zi_b, zj_b, out_v, z_sh,
                   si_a, sj_a, si_b, sj_b):
    w = _wid()
    s = lax.axis_index("s")
    pltpu.sync_copy(z_hbm.at[pl.ds(s * OROWS, OROWS)],
                    z_sh.at[pl.ds(s * OROWS, OROWS)])

    @pl.when(s == NS - 1)
    def _tail():
        pltpu.sync_copy(z_hbm.at[pl.ds(NS * OROWS, TAIL)],
                        z_sh.at[pl.ds(NS * OROWS, TAIL)])

    pltpu.sync_copy(e0_hbm.at[w], e0_v)
    pltpu.sync_copy(e1_hbm.at[w], e1_v)
    plsc.subcore_barrier()

    def issue(j, zi, zj, si, sj):
        pltpu.async_copy(z_sh.at[e0_v.at[j]], zi, si)
        pltpu.async_copy(z_sh.at[e1_v.at[j]], zj, sj)

    def wait(j, zi, zj, si, sj):
        pltpu.make_async_copy(z_sh.at[e0_v.at[j]], zi, si).wait()
        pltpu.make_async_copy(z_sh.at[e1_v.at[j]], zj, sj).wait()

    def compute(j, zi, zj):
        lanes = lax.iota(jnp.int32, L)
        for r in range(DCH // L):
            evec = lanes + (r * L)
            zf = jnp.zeros((L,), jnp.float32)

            def fblk(b, carry):
                a0, a1, a2, a3 = carry
                accs = [a0, a1, a2, a3]
                for k in range(16):
                    col = (lanes + (16 * b + k)) & (DO - 1)
                    gi = plsc.load_gather(zi, [evec, col])
                    gj = plsc.load_gather(zj, [evec, col])
                    accs[k % 4] = accs[k % 4] + gi * gj
                return (accs[0], accs[1], accs[2], accs[3])

            a0, a1, a2, a3 = lax.fori_loop(
                0, DO // 16, fblk, (zf, zf, zf, zf))
            acc = (a0 + a1) + (a2 + a3)
            sig = 1.0 / (1.0 + jnp.exp(-acc))
            out_v[pl.ds(j * DCH + r * L, L)] = sig

    issue(0, zi_a, zj_a, si_a, sj_a)

    def body(t, carry):
        j = 2 * t
        issue(j + 1, zi_b, zj_b, si_b, sj_b)
        wait(j, zi_a, zj_a, si_a, sj_a)
        compute(j, zi_a, zj_a)
        issue(j + 2, zi_a, zj_a, si_a, sj_a)
        wait(j + 1, zi_b, zj_b, si_b, sj_b)
        compute(j + 1, zi_b, zj_b)
        return carry

    lax.fori_loop(0, (DNCH - 1) // 2, body, 0)
    wait(DNCH - 1, zi_a, zj_a, si_a, sj_a)
    compute(DNCH - 1, zi_a, zj_a)
    pltpu.sync_copy(out_v, out_hbm.at[pl.ds(w * EPT, EPT)])


BN = 2000


def _tc_prescale_body(degp_ref, x_ref, xss_ref, dinv_ref):
    deg = jnp.sum(degp_ref[...], axis=1, keepdims=True) + 1.0
    dinv = 1.0 / jnp.sqrt(deg)
    dinv_ref[...] = dinv
    xs = x_ref[...] * dinv
    xss_ref[0] = xs[:, :DH]
    xss_ref[1] = xs[:, DH:]


def _tc_prescale(degp_t, x):
    return pl.pallas_call(
        _tc_prescale_body,
        grid=(N // BN,),
        in_specs=[
            pl.BlockSpec((BN, NW), lambda i: (i, 0)),
            pl.BlockSpec((BN, D), lambda i: (i, 0)),
        ],
        out_specs=[
            pl.BlockSpec((NC, BN, DH), lambda i: (0, i, 0)),
            pl.BlockSpec((BN, 1), lambda i: (i, 0)),
        ],
        out_shape=[
            jax.ShapeDtypeStruct((NC, N, DH), jnp.float32),
            jax.ShapeDtypeStruct((N, 1), jnp.float32),
        ],
    )(degp_t, x)


def _tc_encode_body(s1p_ref, xss_ref, dinv_ref, w1_ref, b1_ref, hss_ref):
    s1 = jnp.concatenate(
        [s1p_ref[0] + xss_ref[0], s1p_ref[1] + xss_ref[1]], axis=-1)
    aggx = s1 * dinv_ref[...]
    h = jnp.dot(aggx, w1_ref[...], preferred_element_type=jnp.float32)
    h = jnp.maximum(h + b1_ref[...][None, :], 0.0)
    nrm = jnp.sqrt(jnp.sum(h * h, axis=1, keepdims=True))
    h = h / jnp.maximum(nrm, 1e-12)
    hs = h * dinv_ref[...]
    hss_ref[0] = hs[:, :DH]
    hss_ref[1] = hs[:, DH:]


def _tc_encode(s1p, xss, dinv, W1, b1):
    return pl.pallas_call(
        _tc_encode_body,
        grid=(N // BN,),
        in_specs=[
            pl.BlockSpec((NC, BN, DH), lambda i: (0, i, 0)),
            pl.BlockSpec((NC, BN, DH), lambda i: (0, i, 0)),
            pl.BlockSpec((BN, 1), lambda i: (i, 0)),
            pl.BlockSpec((D, D), lambda i: (0, 0)),
            pl.BlockSpec((D,), lambda i: (0,)),
        ],
        out_specs=pl.BlockSpec((NC, BN, DH), lambda i: (0, i, 0)),
        out_shape=jax.ShapeDtypeStruct((NC, N, DH), jnp.float32),
    )(s1p, xss, dinv, W1, b1)


def _tc_head_body(s2p_ref, hss_ref, dinv_ref, wmu_ref, bmu_ref, wls_ref,
                  bls_ref, eps_ref, mu_ref, ls_ref, z_ref):
    p = jnp.concatenate(
        [s2p_ref[0] + hss_ref[0], s2p_ref[1] + hss_ref[1]], axis=-1)
    p = p * dinv_ref[...]
    mu = jnp.dot(p, wmu_ref[...], preferred_element_type=jnp.float32)
    mu = mu + bmu_ref[...][None, :]
    ls = jnp.dot(p, wls_ref[...], preferred_element_type=jnp.float32)
    ls = ls + bls_ref[...][None, :]
    mu_ref[...] = mu
    ls_ref[...] = ls
    std = jnp.exp(jnp.clip(ls, -10.0, 10.0))
    z_ref[...] = mu + eps_ref[...] * std


def _tc_head(s2p, hss, dinv, Wmu, bmu, Wls, bls, eps):
    return pl.pallas_call(
        _tc_head_body,
        grid=(N // BN,),
        in_specs=[
            pl.BlockSpec((NC, BN, DH), lambda i: (0, i, 0)),
            pl.BlockSpec((NC, BN, DH), lambda i: (0, i, 0)),
            pl.BlockSpec((BN, 1), lambda i: (i, 0)),
            pl.BlockSpec((D, DO), lambda i: (0, 0)),
            pl.BlockSpec((DO,), lambda i: (0,)),
            pl.BlockSpec((D, DO), lambda i: (0, 0)),
            pl.BlockSpec((DO,), lambda i: (0,)),
            pl.BlockSpec((BN, DO), lambda i: (i, 0)),
        ],
        out_specs=[
            pl.BlockSpec((BN, DO), lambda i: (i, 0)),
            pl.BlockSpec((BN, DO), lambda i: (i, 0)),
            pl.BlockSpec((BN, DO), lambda i: (i, 0)),
        ],
        out_shape=[
            jax.ShapeDtypeStruct((N, DO), jnp.float32),
            jax.ShapeDtypeStruct((N, DO), jnp.float32),
            jax.ShapeDtypeStruct((N, DO), jnp.float32),
        ],
    )(s2p, hss, dinv, Wmu, bmu, Wls, bls, eps)


def kernel(x, edge_index, W1, b1, Wmu, bmu, Wls, bls, eps):
    e0 = edge_index[0]
    e1 = edge_index[1]

    pad = EPC_PAD - EPC
    srcp = jnp.pad(e0.reshape(NS, EPC), ((0, 0), (0, pad)),
                   constant_values=0).reshape(NS, NCHS, CH)
    dstp = jnp.pad(e1.reshape(NS, EPC), ((0, 0), (0, pad)),
                   constant_values=N).reshape(NS, NCHS, CH)
    e0d = e0.reshape(NW, DNCH, DCH)
    e1d = e1.reshape(NW, DNCH, DCH)

    zeros_n = jnp.zeros((N,), jnp.float32)
    zslab = jnp.zeros((ZROWS, DH), jnp.float32)

    degp = _deg_kernel(e1, zeros_n)
    xss, dinv = _tc_prescale(degp.reshape(NW, N).T, x)

    s1p = _scatter_kernel(xss, srcp, dstp, zslab)
    hss = _tc_encode(s1p, xss, dinv, W1, b1)

    s2p = _scatter_kernel(hss, srcp, dstp, zslab)
    mu, logstd, z = _tc_head(s2p, hss, dinv, Wmu, bmu, Wls, bls, eps)

    adj = _decode_kernel(z, e0d, e1d)
    return adj, mu, logstd


# speedup vs baseline: 1.1258x; 1.1258x over previous
import functools

import jax
import jax.numpy as jnp
from jax import lax
from jax.experimental import pallas as pl
from jax.experimental.pallas import tpu as pltpu
from jax.experimental.pallas import tpu_sc as plsc

N = 10000
E = 320000
D = 128
DO = 64
DH = 64

NC = 2
NS = 16
NW = NC * NS
L = 16

EPC = E // NS
CH = 128
NCHS = -(-EPC // CH)
EPC_PAD = NCHS * CH
ZROWS = 632
ACC_N = NS * ZROWS
OROWS = 624
TAIL = N - NS * OROWS

EPT = E // NW
DCH = 80
DNCH = EPT // DCH

_MESH = plsc.VectorSubcoreMesh(
    core_axis_name="c", subcore_axis_name="s", num_cores=NC, num_subcores=NS)


def _wid():
    return lax.axis_index("c") * NS + lax.axis_index("s")


@functools.partial(
    pl.kernel,
    out_type=jax.ShapeDtypeStruct((NW * N,), jnp.float32),
    mesh=_MESH,
    compiler_params=pltpu.CompilerParams(needs_layout_passes=False, use_tc_tiling_on_sc=False),
    scratch_types=[
        pltpu.VMEM((EPT,), jnp.int32),
        pltpu.VMEM((N,), jnp.float32),
    ],
)
def _deg_kernel(dst_hbm, zeros_hbm, out_hbm, dst_v, hist_v):
    w = _wid()
    pltpu.sync_copy(dst_hbm.at[pl.ds(w * EPT, EPT)], dst_v)
    pltpu.sync_copy(zeros_hbm, hist_v)
    ones = jnp.full((L,), 1.0, dtype=jnp.float32)

    def body(i, carry):
        idx = dst_v[pl.ds(i * L, L)]
        plsc.addupdate_scatter(hist_v, [idx], ones)
        return carry

    lax.fori_loop(0, EPT // L, body, 0)
    pltpu.sync_copy(hist_v, out_hbm.at[pl.ds(w * N, N)])


@functools.partial(
    pl.kernel,
    out_type=jax.ShapeDtypeStruct((NC, N, DH), jnp.float32),
    mesh=_MESH,
    compiler_params=pltpu.CompilerParams(use_tc_tiling_on_sc=False),
    scratch_types=[
        pltpu.VMEM((NCHS, CH), jnp.int32),
        pltpu.VMEM((NCHS, CH), jnp.int32),
        pltpu.VMEM((CH, DH), jnp.float32),
        pltpu.VMEM((CH, DH), jnp.float32),
        pltpu.VMEM((CH, DH), jnp.float32),
        pltpu.VMEM((CH, DH), jnp.float32),
        pltpu.VMEM_SHARED((ACC_N, DH), jnp.float32),
        pltpu.SemaphoreType.DMA,
        pltpu.SemaphoreType.DMA,
        pltpu.SemaphoreType.DMA,
        pltpu.SemaphoreType.DMA,
        pltpu.SemaphoreType.DMA,
        pltpu.SemaphoreType.DMA,
        pltpu.SemaphoreType.DMA,
        pltpu.SemaphoreType.DMA,
    ],
)
def _scatter_kernel(vs_hbm, srcp_hbm, dstp_hbm, zslab_hbm, out_hbm,
                    src_v, dst_v, rows0, rows1, rows2, rows3, acc_sh,
                    gsem0, gsem1, gsem2, gsem3,
                    ssem0, ssem1, ssem2, ssem3):
    c = lax.axis_index("c")
    s = lax.axis_index("s")
    pltpu.sync_copy(zslab_hbm, acc_sh.at[pl.ds(s * ZROWS, ZROWS)])
    pltpu.sync_copy(srcp_hbm.at[s], src_v)
    pltpu.sync_copy(dstp_hbm.at[s], dst_v)
    plsc.subcore_barrier()

    NB = 4
    vhalf = vs_hbm.at[c]
    bufs = (rows0, rows1, rows2, rows3)
    gsems = (gsem0, gsem1, gsem2, gsem3)
    ssems = (ssem0, ssem1, ssem2, ssem3)
    gdesc = [None] * NB
    sdesc = [None] * NB
    for j in range(NB - 1):
        gdesc[j] = pltpu.async_copy(
            vhalf.at[src_v.at[j]], bufs[j], gsems[j])
    for j in range(NCHS):
        p = j % NB
        q = (j + NB - 1) % NB
        if j + NB - 1 < NCHS:
            if j >= 1:
                sdesc[q].wait()
            gdesc[q] = pltpu.async_copy(
                vhalf.at[src_v.at[j + NB - 1]], bufs[q], gsems[q])
        gdesc[p].wait()
        sdesc[p] = pltpu.async_copy(
            bufs[p], acc_sh.at[dst_v.at[j]], ssems[p], add=True)
    for j in range(NB):
        if sdesc[j] is not None:
            sdesc[j].wait()

    plsc.subcore_barrier()
    pltpu.sync_copy(acc_sh.at[pl.ds(s * OROWS, OROWS)],
                    out_hbm.at[c, pl.ds(s * OROWS, OROWS)])

    @pl.when(s == NS - 1)
    def _tail():
        pltpu.sync_copy(acc_sh.at[pl.ds(NS * OROWS, TAIL)],
                        out_hbm.at[c, pl.ds(NS * OROWS, TAIL)])


@functools.partial(
    pl.kernel,
    out_type=jax.ShapeDtypeStruct((E,), jnp.float32),
    mesh=_MESH,
    compiler_params=pltpu.CompilerParams(needs_layout_passes=False, use_tc_tiling_on_sc=False),
    scratch_types=[
        pltpu.VMEM((DNCH, DCH), jnp.int32),
        pltpu.VMEM((DNCH, DCH), jnp.int32),
        pltpu.VMEM((DCH, DO), jnp.float32),
        pltpu.VMEM((DCH, DO), jnp.float32),
        pltpu.VMEM((DCH, DO), jnp.float32),
        pltpu.VMEM((DCH, DO), jnp.float32),
        pltpu.VMEM((EPT,), jnp.float32),
        pltpu.VMEM_SHARED((N, DO), jnp.float32),
        pltpu.SemaphoreType.DMA,
        pltpu.SemaphoreType.DMA,
        pltpu.SemaphoreType.DMA,
        pltpu.SemaphoreType.DMA,
    ],
)
def _decode_kernel(z_hbm, e0_hbm, e1_hbm, out_hbm,
                   e0_v, e1_v, zi_a, zj_a, zi_b, zj_b, out_v, z_sh,
                   si_a, sj_a, si_b, sj_b):
    w = _wid()
    s = lax.axis_index("s")
    pltpu.sync_copy(z_hbm.at[pl.ds(s * OROWS, OROWS)],
                    z_sh.at[pl.ds(s * OROWS, OROWS)])

    @pl.when(s == NS - 1)
    def _tail():
        pltpu.sync_copy(z_hbm.at[pl.ds(NS * OROWS, TAIL)],
                        z_sh.at[pl.ds(NS * OROWS, TAIL)])

    pltpu.sync_copy(e0_hbm.at[w], e0_v)
    pltpu.sync_copy(e1_hbm.at[w], e1_v)
    plsc.subcore_barrier()

    def issue(j, zi, zj, si, sj):
        pltpu.async_copy(z_sh.at[e0_v.at[j]], zi, si)
        pltpu.async_copy(z_sh.at[e1_v.at[j]], zj, sj)

    def wait(j, zi, zj, si, sj):
        pltpu.make_async_copy(z_sh.at[e0_v.at[j]], zi, si).wait()
        pltpu.make_async_copy(z_sh.at[e1_v.at[j]], zj, sj).wait()

    def compute(j, zi, zj):
        lanes = lax.iota(jnp.int32, L)
        for r in range(DCH // L):
            evec = lanes + (r * L)
            zf = jnp.zeros((L,), jnp.float32)

            def fblk(b, carry):
                a0, a1, a2, a3 = carry
                accs = [a0, a1, a2, a3]
                for k in range(16):
                    col = (lanes + (16 * b + k)) & (DO - 1)
                    gi = plsc.load_gather(zi, [evec, col])
                    gj = plsc.load_gather(zj, [evec, col])
                    accs[k % 4] = accs[k % 4] + gi * gj
                return (accs[0], accs[1], accs[2], accs[3])

            a0, a1, a2, a3 = lax.fori_loop(
                0, DO // 16, fblk, (zf, zf, zf, zf))
            acc = (a0 + a1) + (a2 + a3)
            sig = 1.0 / (1.0 + jnp.exp(-acc))
            out_v[pl.ds(j * DCH + r * L, L)] = sig

    issue(0, zi_a, zj_a, si_a, sj_a)

    def body(t, carry):
        j = 2 * t
        issue(j + 1, zi_b, zj_b, si_b, sj_b)
        wait(j, zi_a, zj_a, si_a, sj_a)
        compute(j, zi_a, zj_a)
        issue(j + 2, zi_a, zj_a, si_a, sj_a)
        wait(j + 1, zi_b, zj_b, si_b, sj_b)
        compute(j + 1, zi_b, zj_b)
        return carry

    lax.fori_loop(0, (DNCH - 1) // 2, body, 0)
    wait(DNCH - 1, zi_a, zj_a, si_a, sj_a)
    compute(DNCH - 1, zi_a, zj_a)
    pltpu.sync_copy(out_v, out_hbm.at[pl.ds(w * EPT, EPT)])


BN = 2000


def _tc_prescale_body(degp_ref, x_ref, xss_ref, dinv_ref):
    deg = jnp.sum(degp_ref[...], axis=1, keepdims=True) + 1.0
    dinv = 1.0 / jnp.sqrt(deg)
    dinv_ref[...] = dinv
    xs = x_ref[...] * dinv
    xss_ref[0] = xs[:, :DH]
    xss_ref[1] = xs[:, DH:]


def _tc_prescale(degp_t, x):
    return pl.pallas_call(
        _tc_prescale_body,
        grid=(N // BN,),
        in_specs=[
            pl.BlockSpec((BN, NW), lambda i: (i, 0)),
            pl.BlockSpec((BN, D), lambda i: (i, 0)),
        ],
        out_specs=[
            pl.BlockSpec((NC, BN, DH), lambda i: (0, i, 0)),
            pl.BlockSpec((BN, 1), lambda i: (i, 0)),
        ],
        out_shape=[
            jax.ShapeDtypeStruct((NC, N, DH), jnp.float32),
            jax.ShapeDtypeStruct((N, 1), jnp.float32),
        ],
    )(degp_t, x)


def _tc_encode_body(s1p_ref, xss_ref, dinv_ref, w1_ref, b1_ref, hss_ref):
    s1 = jnp.concatenate(
        [s1p_ref[0] + xss_ref[0], s1p_ref[1] + xss_ref[1]], axis=-1)
    aggx = s1 * dinv_ref[...]
    h = jnp.dot(aggx, w1_ref[...], preferred_element_type=jnp.float32)
    h = jnp.maximum(h + b1_ref[...][None, :], 0.0)
    nrm = jnp.sqrt(jnp.sum(h * h, axis=1, keepdims=True))
    h = h / jnp.maximum(nrm, 1e-12)
    hs = h * dinv_ref[...]
    hss_ref[0] = hs[:, :DH]
    hss_ref[1] = hs[:, DH:]


def _tc_encode(s1p, xss, dinv, W1, b1):
    return pl.pallas_call(
        _tc_encode_body,
        grid=(N // BN,),
        in_specs=[
            pl.BlockSpec((NC, BN, DH), lambda i: (0, i, 0)),
            pl.BlockSpec((NC, BN, DH), lambda i: (0, i, 0)),
            pl.BlockSpec((BN, 1), lambda i: (i, 0)),
            pl.BlockSpec((D, D), lambda i: (0, 0)),
            pl.BlockSpec((D,), lambda i: (0,)),
        ],
        out_specs=pl.BlockSpec((NC, BN, DH), lambda i: (0, i, 0)),
        out_shape=jax.ShapeDtypeStruct((NC, N, DH), jnp.float32),
    )(s1p, xss, dinv, W1, b1)


def _tc_head_body(s2p_ref, hss_ref, dinv_ref, wmu_ref, bmu_ref, wls_ref,
                  bls_ref, eps_ref, mu_ref, ls_ref, z_ref):
    p = jnp.concatenate(
        [s2p_ref[0] + hss_ref[0], s2p_ref[1] + hss_ref[1]], axis=-1)
    p = p * dinv_ref[...]
    mu = jnp.dot(p, wmu_ref[...], preferred_element_type=jnp.float32)
    mu = mu + bmu_ref[...][None, :]
    ls = jnp.dot(p, wls_ref[...], preferred_element_type=jnp.float32)
    ls = ls + bls_ref[...][None, :]
    mu_ref[...] = mu
    ls_ref[...] = ls
    std = jnp.exp(jnp.clip(ls, -10.0, 10.0))
    z_ref[...] = mu + eps_ref[...] * std


def _tc_head(s2p, hss, dinv, Wmu, bmu, Wls, bls, eps):
    return pl.pallas_call(
        _tc_head_body,
        grid=(N // BN,),
        in_specs=[
            pl.BlockSpec((NC, BN, DH), lambda i: (0, i, 0)),
            pl.BlockSpec((NC, BN, DH), lambda i: (0, i, 0)),
            pl.BlockSpec((BN, 1), lambda i: (i, 0)),
            pl.BlockSpec((D, DO), lambda i: (0, 0)),
            pl.BlockSpec((DO,), lambda i: (0,)),
            pl.BlockSpec((D, DO), lambda i: (0, 0)),
            pl.BlockSpec((DO,), lambda i: (0,)),
            pl.BlockSpec((BN, DO), lambda i: (i, 0)),
        ],
        out_specs=[
            pl.BlockSpec((BN, DO), lambda i: (i, 0)),
            pl.BlockSpec((BN, DO), lambda i: (i, 0)),
            pl.BlockSpec((BN, DO), lambda i: (i, 0)),
        ],
        out_shape=[
            jax.ShapeDtypeStruct((N, DO), jnp.float32),
            jax.ShapeDtypeStruct((N, DO), jnp.float32),
            jax.ShapeDtypeStruct((N, DO), jnp.float32),
        ],
    )(s2p, hss, dinv, Wmu, bmu, Wls, bls, eps)


def kernel(x, edge_index, W1, b1, Wmu, bmu, Wls, bls, eps):
    e0 = edge_index[0]
    e1 = edge_index[1]

    pad = EPC_PAD - EPC
    srcp = jnp.pad(e0.reshape(NS, EPC), ((0, 0), (0, pad)),
                   constant_values=0).reshape(NS, NCHS, CH)
    dstp = jnp.pad(e1.reshape(NS, EPC), ((0, 0), (0, pad)),
                   constant_values=N).reshape(NS, NCHS, CH)
    e0d = e0.reshape(NW, DNCH, DCH)
    e1d = e1.reshape(NW, DNCH, DCH)

    zeros_n = jnp.zeros((N,), jnp.float32)
    zslab = jnp.zeros((ZROWS, DH), jnp.float32)

    degp = _deg_kernel(e1, zeros_n)
    xss, dinv = _tc_prescale(degp.reshape(NW, N).T, x)

    s1p = _scatter_kernel(xss, srcp, dstp, zslab)
    hss = _tc_encode(s1p, xss, dinv, W1, b1)

    s2p = _scatter_kernel(hss, srcp, dstp, zslab)
    mu, logstd, z = _tc_head(s2p, hss, dinv, Wmu, bmu, Wls, bls, eps)

    adj = _decode_kernel(z, e0d, e1d)
    return adj, mu, logstd


# speedup vs baseline: 1.1275x; 1.0015x over previous
import functools

import jax
import jax.numpy as jnp
from jax import lax
from jax.experimental import pallas as pl
from jax.experimental.pallas import tpu as pltpu
from jax.experimental.pallas import tpu_sc as plsc

N = 10000
E = 320000
D = 128
DO = 64
DH = 64

NC = 2
NS = 16
NW = NC * NS
L = 16

EPC = E // NS
CH = 128
NCHS = -(-EPC // CH)
EPC_PAD = NCHS * CH
ZROWS = 632
ACC_N = NS * ZROWS
OROWS = 624
TAIL = N - NS * OROWS

EPT = E // NW
DCH = 80
DNCH = EPT // DCH

_MESH = plsc.VectorSubcoreMesh(
    core_axis_name="c", subcore_axis_name="s", num_cores=NC, num_subcores=NS)


def _wid():
    return lax.axis_index("c") * NS + lax.axis_index("s")


@functools.partial(
    pl.kernel,
    out_type=jax.ShapeDtypeStruct((NW * N,), jnp.float32),
    mesh=_MESH,
    compiler_params=pltpu.CompilerParams(needs_layout_passes=False, use_tc_tiling_on_sc=False),
    scratch_types=[
        pltpu.VMEM((EPT,), jnp.int32),
        pltpu.VMEM((N,), jnp.float32),
    ],
)
def _deg_kernel(dst_hbm, zeros_hbm, out_hbm, dst_v, hist_v):
    w = _wid()
    pltpu.sync_copy(dst_hbm.at[pl.ds(w * EPT, EPT)], dst_v)
    pltpu.sync_copy(zeros_hbm, hist_v)
    ones = jnp.full((L,), 1.0, dtype=jnp.float32)

    def body(i, carry):
        idx = dst_v[pl.ds(i * L, L)]
        plsc.addupdate_scatter(hist_v, [idx], ones)
        return carry

    lax.fori_loop(0, EPT // L, body, 0)
    pltpu.sync_copy(hist_v, out_hbm.at[pl.ds(w * N, N)])


@functools.partial(
    pl.kernel,
    out_type=jax.ShapeDtypeStruct((NC, N, DH), jnp.float32),
    mesh=_MESH,
    compiler_params=pltpu.CompilerParams(use_tc_tiling_on_sc=False),
    scratch_types=[
        pltpu.VMEM((NCHS, CH), jnp.int32),
        pltpu.VMEM((NCHS, CH), jnp.int32),
        pltpu.VMEM((CH, DH), jnp.float32),
        pltpu.VMEM((CH, DH), jnp.float32),
        pltpu.VMEM((CH, DH), jnp.float32),
        pltpu.VMEM((CH, DH), jnp.float32),
        pltpu.VMEM((CH, DH), jnp.float32),
        pltpu.VMEM((CH, DH), jnp.float32),
        pltpu.VMEM((CH, DH), jnp.float32),
        pltpu.VMEM((CH, DH), jnp.float32),
        pltpu.VMEM_SHARED((ACC_N, DH), jnp.float32),
        pltpu.SemaphoreType.DMA,
        pltpu.SemaphoreType.DMA,
        pltpu.SemaphoreType.DMA,
        pltpu.SemaphoreType.DMA,
        pltpu.SemaphoreType.DMA,
        pltpu.SemaphoreType.DMA,
        pltpu.SemaphoreType.DMA,
        pltpu.SemaphoreType.DMA,
        pltpu.SemaphoreType.DMA,
        pltpu.SemaphoreType.DMA,
        pltpu.SemaphoreType.DMA,
        pltpu.SemaphoreType.DMA,
        pltpu.SemaphoreType.DMA,
        pltpu.SemaphoreType.DMA,
        pltpu.SemaphoreType.DMA,
        pltpu.SemaphoreType.DMA,
    ],
)
def _scatter_kernel(vs_hbm, srcp_hbm, dstp_hbm, zslab_hbm, out_hbm,
                    src_v, dst_v, rows0, rows1, rows2, rows3,
                    rows4, rows5, rows6, rows7, acc_sh,
                    gsem0, gsem1, gsem2, gsem3,
                    gsem4, gsem5, gsem6, gsem7,
                    ssem0, ssem1, ssem2, ssem3,
                    ssem4, ssem5, ssem6, ssem7):
    c = lax.axis_index("c")
    s = lax.axis_index("s")
    pltpu.sync_copy(zslab_hbm, acc_sh.at[pl.ds(s * ZROWS, ZROWS)])
    pltpu.sync_copy(srcp_hbm.at[s], src_v)
    pltpu.sync_copy(dstp_hbm.at[s], dst_v)
    plsc.subcore_barrier()

    NB = 6
    vhalf = vs_hbm.at[c]
    bufs = (rows0, rows1, rows2, rows3, rows4, rows5, rows6, rows7)
    gsems = (gsem0, gsem1, gsem2, gsem3, gsem4, gsem5, gsem6, gsem7)
    ssems = (ssem0, ssem1, ssem2, ssem3, ssem4, ssem5, ssem6, ssem7)
    gdesc = [None] * NB
    sdesc = [None] * NB
    for j in range(NB - 1):
        gdesc[j] = pltpu.async_copy(
            vhalf.at[src_v.at[j]], bufs[j], gsems[j])
    for j in range(NCHS):
        p = j % NB
        q = (j + NB - 1) % NB
        if j + NB - 1 < NCHS:
            if j >= 1:
                sdesc[q].wait()
            gdesc[q] = pltpu.async_copy(
                vhalf.at[src_v.at[j + NB - 1]], bufs[q], gsems[q])
        gdesc[p].wait()
        sdesc[p] = pltpu.async_copy(
            bufs[p], acc_sh.at[dst_v.at[j]], ssems[p], add=True)
    for j in range(NB):
        if sdesc[j] is not None:
            sdesc[j].wait()

    plsc.subcore_barrier()
    pltpu.sync_copy(acc_sh.at[pl.ds(s * OROWS, OROWS)],
                    out_hbm.at[c, pl.ds(s * OROWS, OROWS)])

    @pl.when(s == NS - 1)
    def _tail():
        pltpu.sync_copy(acc_sh.at[pl.ds(NS * OROWS, TAIL)],
                        out_hbm.at[c, pl.ds(NS * OROWS, TAIL)])


@functools.partial(
    pl.kernel,
    out_type=jax.ShapeDtypeStruct((E,), jnp.float32),
    mesh=_MESH,
    compiler_params=pltpu.CompilerParams(needs_layout_passes=False, use_tc_tiling_on_sc=False),
    scratch_types=[
        pltpu.VMEM((DNCH, DCH), jnp.int32),
        pltpu.VMEM((DNCH, DCH), jnp.int32),
        pltpu.VMEM((DCH, DO), jnp.float32),
        pltpu.VMEM((DCH, DO), jnp.float32),
        pltpu.VMEM((DCH, DO), jnp.float32),
        pltpu.VMEM((DCH, DO), jnp.float32),
        pltpu.VMEM((EPT,), jnp.float32),
        pltpu.VMEM_SHARED((N, DO), jnp.float32),
        pltpu.SemaphoreType.DMA,
        pltpu.SemaphoreType.DMA,
        pltpu.SemaphoreType.DMA,
        pltpu.SemaphoreType.DMA,
    ],
)
def _decode_kernel(z_hbm, e0_hbm, e1_hbm, out_hbm,
                   e0_v, e1_v, zi_a, zj_a, zi_b, zj_b, out_v, z_sh,
                   si_a, sj_a, si_b, sj_b):
    w = _wid()
    s = lax.axis_index("s")
    pltpu.sync_copy(z_hbm.at[pl.ds(s * OROWS, OROWS)],
                    z_sh.at[pl.ds(s * OROWS, OROWS)])

    @pl.when(s == NS - 1)
    def _tail():
        pltpu.sync_copy(z_hbm.at[pl.ds(NS * OROWS, TAIL)],
                        z_sh.at[pl.ds(NS * OROWS, TAIL)])

    pltpu.sync_copy(e0_hbm.at[w], e0_v)
    pltpu.sync_copy(e1_hbm.at[w], e1_v)
    plsc.subcore_barrier()

    def issue(j, zi, zj, si, sj):
        pltpu.async_copy(z_sh.at[e0_v.at[j]], zi, si)
        pltpu.async_copy(z_sh.at[e1_v.at[j]], zj, sj)

    def wait(j, zi, zj, si, sj):
        pltpu.make_async_copy(z_sh.at[e0_v.at[j]], zi, si).wait()
        pltpu.make_async_copy(z_sh.at[e1_v.at[j]], zj, sj).wait()

    def compute(j, zi, zj):
        lanes = lax.iota(jnp.int32, L)
        for r in range(DCH // L):
            evec = lanes + (r * L)
            zf = jnp.zeros((L,), jnp.float32)

            def fblk(b, carry):
                a0, a1, a2, a3 = carry
                accs = [a0, a1, a2, a3]
                for k in range(16):
                    col = (lanes + (16 * b + k)) & (DO - 1)
                    gi = plsc.load_gather(zi, [evec, col])
                    gj = plsc.load_gather(zj, [evec, col])
                    accs[k % 4] = accs[k % 4] + gi * gj
                return (accs[0], accs[1], accs[2], accs[3])

            a0, a1, a2, a3 = lax.fori_loop(
                0, DO // 16, fblk, (zf, zf, zf, zf))
            acc = (a0 + a1) + (a2 + a3)
            sig = 1.0 / (1.0 + jnp.exp(-acc))
            out_v[pl.ds(j * DCH + r * L, L)] = sig

    issue(0, zi_a, zj_a, si_a, sj_a)

    def body(t, carry):
        j = 2 * t
        issue(j + 1, zi_b, zj_b, si_b, sj_b)
        wait(j, zi_a, zj_a, si_a, sj_a)
        compute(j, zi_a, zj_a)
        issue(j + 2, zi_a, zj_a, si_a, sj_a)
        wait(j + 1, zi_b, zj_b, si_b, sj_b)
        compute(j + 1, zi_b, zj_b)
        return carry

    lax.fori_loop(0, (DNCH - 1) // 2, body, 0)
    wait(DNCH - 1, zi_a, zj_a, si_a, sj_a)
    compute(DNCH - 1, zi_a, zj_a)
    pltpu.sync_copy(out_v, out_hbm.at[pl.ds(w * EPT, EPT)])


BN = 2000


def _tc_prescale_body(degp_ref, x_ref, xss_ref, dinv_ref):
    deg = jnp.sum(degp_ref[...], axis=1, keepdims=True) + 1.0
    dinv = 1.0 / jnp.sqrt(deg)
    dinv_ref[...] = dinv
    xs = x_ref[...] * dinv
    xss_ref[0] = xs[:, :DH]
    xss_ref[1] = xs[:, DH:]


def _tc_prescale(degp_t, x):
    return pl.pallas_call(
        _tc_prescale_body,
        grid=(N // BN,),
        in_specs=[
            pl.BlockSpec((BN, NW), lambda i: (i, 0)),
            pl.BlockSpec((BN, D), lambda i: (i, 0)),
        ],
        out_specs=[
            pl.BlockSpec((NC, BN, DH), lambda i: (0, i, 0)),
            pl.BlockSpec((BN, 1), lambda i: (i, 0)),
        ],
        out_shape=[
            jax.ShapeDtypeStruct((NC, N, DH), jnp.float32),
            jax.ShapeDtypeStruct((N, 1), jnp.float32),
        ],
    )(degp_t, x)


def _tc_encode_body(s1p_ref, xss_ref, dinv_ref, w1_ref, b1_ref, hss_ref):
    s1 = jnp.concatenate(
        [s1p_ref[0] + xss_ref[0], s1p_ref[1] + xss_ref[1]], axis=-1)
    aggx = s1 * dinv_ref[...]
    h = jnp.dot(aggx, w1_ref[...], preferred_element_type=jnp.float32)
    h = jnp.maximum(h + b1_ref[...][None, :], 0.0)
    nrm = jnp.sqrt(jnp.sum(h * h, axis=1, keepdims=True))
    h = h / jnp.maximum(nrm, 1e-12)
    hs = h * dinv_ref[...]
    hss_ref[0] = hs[:, :DH]
    hss_ref[1] = hs[:, DH:]


def _tc_encode(s1p, xss, dinv, W1, b1):
    return pl.pallas_call(
        _tc_encode_body,
        grid=(N // BN,),
        in_specs=[
            pl.BlockSpec((NC, BN, DH), lambda i: (0, i, 0)),
            pl.BlockSpec((NC, BN, DH), lambda i: (0, i, 0)),
            pl.BlockSpec((BN, 1), lambda i: (i, 0)),
            pl.BlockSpec((D, D), lambda i: (0, 0)),
            pl.BlockSpec((D,), lambda i: (0,)),
        ],
        out_specs=pl.BlockSpec((NC, BN, DH), lambda i: (0, i, 0)),
        out_shape=jax.ShapeDtypeStruct((NC, N, DH), jnp.float32),
    )(s1p, xss, dinv, W1, b1)


def _tc_head_body(s2p_ref, hss_ref, dinv_ref, wmu_ref, bmu_ref, wls_ref,
                  bls_ref, eps_ref, mu_ref, ls_ref, z_ref):
    p = jnp.concatenate(
        [s2p_ref[0] + hss_ref[0], s2p_ref[1] + hss_ref[1]], axis=-1)
    p = p * dinv_ref[...]
    mu = jnp.dot(p, wmu_ref[...], preferred_element_type=jnp.float32)
    mu = mu + bmu_ref[...][None, :]
    ls = jnp.dot(p, wls_ref[...], preferred_element_type=jnp.float32)
    ls = ls + bls_ref[...][None, :]
    mu_ref[...] = mu
    ls_ref[...] = ls
    std = jnp.exp(jnp.clip(ls, -10.0, 10.0))
    z_ref[...] = mu + eps_ref[...] * std


def _tc_head(s2p, hss, dinv, Wmu, bmu, Wls, bls, eps):
    return pl.pallas_call(
        _tc_head_body,
        grid=(N // BN,),
        in_specs=[
            pl.BlockSpec((NC, BN, DH), lambda i: (0, i, 0)),
            pl.BlockSpec((NC, BN, DH), lambda i: (0, i, 0)),
            pl.BlockSpec((BN, 1), lambda i: (i, 0)),
            pl.BlockSpec((D, DO), lambda i: (0, 0)),
            pl.BlockSpec((DO,), lambda i: (0,)),
            pl.BlockSpec((D, DO), lambda i: (0, 0)),
            pl.BlockSpec((DO,), lambda i: (0,)),
            pl.BlockSpec((BN, DO), lambda i: (i, 0)),
        ],
        out_specs=[
            pl.BlockSpec((BN, DO), lambda i: (i, 0)),
            pl.BlockSpec((BN, DO), lambda i: (i, 0)),
            pl.BlockSpec((BN, DO), lambda i: (i, 0)),
        ],
        out_shape=[
            jax.ShapeDtypeStruct((N, DO), jnp.float32),
            jax.ShapeDtypeStruct((N, DO), jnp.float32),
            jax.ShapeDtypeStruct((N, DO), jnp.float32),
        ],
    )(s2p, hss, dinv, Wmu, bmu, Wls, bls, eps)


def kernel(x, edge_index, W1, b1, Wmu, bmu, Wls, bls, eps):
    e0 = edge_index[0]
    e1 = edge_index[1]

    pad = EPC_PAD - EPC
    srcp = jnp.pad(e0.reshape(NS, EPC), ((0, 0), (0, pad)),
                   constant_values=0).reshape(NS, NCHS, CH)
    dstp = jnp.pad(e1.reshape(NS, EPC), ((0, 0), (0, pad)),
                   constant_values=N).reshape(NS, NCHS, CH)
    e0d = e0.reshape(NW, DNCH, DCH)
    e1d = e1.reshape(NW, DNCH, DCH)

    zeros_n = jnp.zeros((N,), jnp.float32)
    zslab = jnp.zeros((ZROWS, DH), jnp.float32)

    degp = _deg_kernel(e1, zeros_n)
    xss, dinv = _tc_prescale(degp.reshape(NW, N).T, x)

    s1p = _scatter_kernel(xss, srcp, dstp, zslab)
    hss = _tc_encode(s1p, xss, dinv, W1, b1)

    s2p = _scatter_kernel(hss, srcp, dstp, zslab)
    mu, logstd, z = _tc_head(s2p, hss, dinv, Wmu, bmu, Wls, bls, eps)

    adj = _decode_kernel(z, e0d, e1d)
    return adj, mu, logstd
